# Initial kernel scaffold; baseline (speedup 1.0000x reference)
import jax
import jax.numpy as jnp
from jax.experimental import pallas as pl


def kernel(embeddings):
    raise NotImplementedError("write your pallas kernel here")


# speedup vs baseline: 4.8396x; 4.8396x over previous
import functools

import jax
import jax.numpy as jnp
from jax.experimental import pallas as pl

_K = 8
_NEG_INF = float("-inf")


def _normalize_body(x_ref, o_ref):
    x = x_ref[...]
    n = jnp.sqrt(jnp.sum(x * x, axis=1, keepdims=True))
    o_ref[...] = x / jnp.maximum(n, 1e-12)


def _simtopk_body(a_ref, b_ref, idx_ref, *, strip: int, n: int):
    a = a_ref[...]
    b = b_ref[...]
    sim = jax.lax.dot_general(
        a, b, (((1,), (1,)), ((), ())), preferred_element_type=jnp.float32)
    i = pl.program_id(0)
    row_ids = jax.lax.broadcasted_iota(jnp.int32, (strip, n), 0) + i * strip
    col_ids = jax.lax.broadcasted_iota(jnp.int32, (strip, n), 1)
    sim = jnp.where(col_ids == row_ids, _NEG_INF, sim)
    cols = []
    for _ in range(_K):
        m = jnp.max(sim, axis=1, keepdims=True)
        amax = jnp.min(jnp.where(sim == m, col_ids, n), axis=1, keepdims=True)
        cols.append(amax)
        sim = jnp.where(col_ids == amax, _NEG_INF, sim)
    idx_ref[...] = jnp.concatenate(cols, axis=1)


def _adjacency_body(idx_strip_ref, idxt_ref, out_ref, *, strip: int, n: int):
    idx_strip = idx_strip_ref[...]
    idxt = idxt_ref[...]
    i0 = pl.program_id(0) * strip
    row_ids = jax.lax.broadcasted_iota(jnp.int32, (strip, n), 0) + i0
    col_ids = jax.lax.broadcasted_iota(jnp.int32, (strip, n), 1)
    acc = jnp.zeros((strip, n), dtype=jnp.bool_)
    for k in range(_K):
        fwd = col_ids == idx_strip[:, k:k + 1]
        bwd = row_ids == idxt[k:k + 1, :]
        acc = acc | fwd | bwd
    a = acc.astype(jnp.float32)
    rs = jnp.maximum(jnp.sum(a, axis=1, keepdims=True), 1e-8)
    out_ref[...] = a / rs


@jax.jit
def kernel(embeddings):
    n, d = embeddings.shape
    rn = min(1024, n)
    xn = pl.pallas_call(
        _normalize_body,
        grid=(n // rn,),
        in_specs=[pl.BlockSpec((rn, d), lambda i: (i, 0))],
        out_specs=pl.BlockSpec((rn, d), lambda i: (i, 0)),
        out_shape=jax.ShapeDtypeStruct((n, d), jnp.float32),
    )(embeddings)

    strip = min(256, n)
    nstrips = n // strip
    idx = pl.pallas_call(
        functools.partial(_simtopk_body, strip=strip, n=n),
        grid=(nstrips,),
        in_specs=[
            pl.BlockSpec((strip, d), lambda i: (i, 0)),
            pl.BlockSpec((n, d), lambda i: (0, 0)),
        ],
        out_specs=pl.BlockSpec((strip, _K), lambda i: (i, 0)),
        out_shape=jax.ShapeDtypeStruct((n, _K), jnp.int32),
    )(xn, xn)

    idxt = idx.T

    adjacency = pl.pallas_call(
        functools.partial(_adjacency_body, strip=strip, n=n),
        grid=(nstrips,),
        in_specs=[
            pl.BlockSpec((strip, _K), lambda i: (i, 0)),
            pl.BlockSpec((_K, n), lambda i: (0, 0)),
        ],
        out_specs=pl.BlockSpec((strip, n), lambda i: (i, 0)),
        out_shape=jax.ShapeDtypeStruct((n, n), jnp.float32),
    )(idx, idxt)

    return adjacency, xn
